# Initial kernel scaffold; baseline (speedup 1.0000x reference)
import jax
import jax.numpy as jnp
from jax.experimental import pallas as pl


def kernel(x, w_gate, W1, b1, W2, b2):
    raise NotImplementedError("write your pallas kernel here")


# speedup vs baseline: 1.1171x; 1.1171x over previous
import jax
import jax.numpy as jnp
from jax.experimental import pallas as pl
from jax.experimental.pallas import tpu as pltpu

B = 4096
D = 1024
H = 2048
E = 8
K = 4
BH = 512
NJ = H // BH


def _gates_kernel(x_ref, wg_ref, b2_ref, gates_ref, y0_ref):
    l = jax.lax.dot_general(
        x_ref[...], wg_ref[...], (((1,), (0,)), ((), ())),
        preferred_element_type=jnp.float32)
    ei = jax.lax.broadcasted_iota(jnp.int32, (B, E), 1)
    rank = jnp.zeros((B, E), jnp.int32)
    for j in range(E):
        lj = l[:, j:j + 1]
        beats = (lj > l) | ((lj == l) & (j < ei))
        rank = rank + beats.astype(jnp.int32)
    sel = rank < K
    m = jnp.max(l, axis=1, keepdims=True)
    ex = jnp.where(sel, jnp.exp(l - m), 0.0)
    g = ex / jnp.sum(ex, axis=1, keepdims=True)
    gates_ref[...] = g
    y0_ref[...] = jnp.dot(g, b2_ref[...], preferred_element_type=jnp.float32)


def _moe_kernel(xb_ref, w1_ref, b1_ref, w2_ref, gates_ref, y0_ref, out_ref):
    e = pl.program_id(0)
    j = pl.program_id(1)

    @pl.when((e == 0) & (j == 0))
    def _init():
        out_ref[...] = y0_ref[...]

    h = jnp.dot(xb_ref[...], w1_ref[0], preferred_element_type=jnp.float32)
    h = jnp.maximum(h + b1_ref[0], 0.0)
    partial = jnp.sum(h * w2_ref[0], axis=1, keepdims=True)
    onehot = (jax.lax.broadcasted_iota(jnp.int32, (E, 1), 0) == e
              ).astype(jnp.float32)
    g = jnp.dot(gates_ref[...], onehot, preferred_element_type=jnp.float32)
    out_ref[...] += g * partial


def kernel(x, w_gate, W1, b1, W2, b2):
    gates, y0 = pl.pallas_call(
        _gates_kernel,
        out_shape=[
            jax.ShapeDtypeStruct((B, E), jnp.float32),
            jax.ShapeDtypeStruct((B, 1), jnp.float32),
        ],
    )(x, w_gate, b2)

    xb = x.astype(jnp.bfloat16)
    W1b = W1.astype(jnp.bfloat16)
    b1r = b1.reshape(E, 1, H)
    W2r = W2.reshape(E, 1, H)

    out = pl.pallas_call(
        _moe_kernel,
        grid=(E, NJ),
        in_specs=[
            pl.BlockSpec((B, D), lambda e, j: (0, 0)),
            pl.BlockSpec((1, D, BH), lambda e, j: (e, 0, j)),
            pl.BlockSpec((1, 1, BH), lambda e, j: (e, 0, j)),
            pl.BlockSpec((1, 1, BH), lambda e, j: (e, 0, j)),
            pl.BlockSpec((B, E), lambda e, j: (0, 0)),
            pl.BlockSpec((B, 1), lambda e, j: (0, 0)),
        ],
        out_specs=pl.BlockSpec((B, 1), lambda e, j: (0, 0)),
        out_shape=jax.ShapeDtypeStruct((B, 1), jnp.float32),
        compiler_params=pltpu.CompilerParams(
            dimension_semantics=("arbitrary", "arbitrary")),
    )(xb, W1b, b1r, W2r, gates, y0)
    return out


# speedup vs baseline: 1.4552x; 1.3026x over previous
import jax
import jax.numpy as jnp
from jax.experimental import pallas as pl
from jax.experimental.pallas import tpu as pltpu

B = 4096
D = 1024
H = 2048
E = 8
K = 4
BH = 1024
NJ = H // BH


def _gates_kernel(x_ref, wg_ref, b2_ref, gates_ref, y0_ref, xb_ref):
    x = x_ref[...]
    l = jax.lax.dot_general(
        x, wg_ref[...], (((1,), (0,)), ((), ())),
        preferred_element_type=jnp.float32)
    lt = l.T
    ei = jax.lax.broadcasted_iota(jnp.int32, (E, B), 0)
    rank = jnp.zeros((E, B), jnp.int32)
    for j in range(E):
        lj = lt[j:j + 1, :]
        beats = (lj > lt) | ((lj == lt) & (j < ei))
        rank = rank + beats.astype(jnp.int32)
    sel = rank < K
    m = jnp.max(lt, axis=0, keepdims=True)
    ex = jnp.where(sel, jnp.exp(lt - m), 0.0)
    g = (ex / jnp.sum(ex, axis=0, keepdims=True)).T
    gates_ref[...] = g
    y0_ref[...] = jnp.dot(g, b2_ref[...], preferred_element_type=jnp.float32)
    xb_ref[...] = x.astype(jnp.bfloat16)


def _moe_kernel(xb_ref, w1_ref, b1_ref, w2_ref, gates_ref, y0_ref, out_ref):
    e = pl.program_id(0)
    j = pl.program_id(1)

    @pl.when((e == 0) & (j == 0))
    def _init():
        out_ref[...] = y0_ref[...]

    w1b = w1_ref[0].astype(jnp.bfloat16)
    h = jnp.dot(xb_ref[...], w1b, preferred_element_type=jnp.float32)
    h = jnp.maximum(h + b1_ref[0], 0.0)
    partial = jnp.sum(h * w2_ref[0], axis=1, keepdims=True)
    onehot = (jax.lax.broadcasted_iota(jnp.int32, (E, 1), 0) == e
              ).astype(jnp.float32)
    g = jnp.dot(gates_ref[...], onehot, preferred_element_type=jnp.float32)
    out_ref[...] += g * partial


def kernel(x, w_gate, W1, b1, W2, b2):
    gates, y0, xb = pl.pallas_call(
        _gates_kernel,
        out_shape=[
            jax.ShapeDtypeStruct((B, E), jnp.float32),
            jax.ShapeDtypeStruct((B, 1), jnp.float32),
            jax.ShapeDtypeStruct((B, D), jnp.bfloat16),
        ],
    )(x, w_gate, b2)

    b1r = b1.reshape(E, 1, H)
    W2r = W2.reshape(E, 1, H)

    out = pl.pallas_call(
        _moe_kernel,
        grid=(E, NJ),
        in_specs=[
            pl.BlockSpec((B, D), lambda e, j: (0, 0)),
            pl.BlockSpec((1, D, BH), lambda e, j: (e, 0, j)),
            pl.BlockSpec((1, 1, BH), lambda e, j: (e, 0, j)),
            pl.BlockSpec((1, 1, BH), lambda e, j: (e, 0, j)),
            pl.BlockSpec((B, E), lambda e, j: (0, 0)),
            pl.BlockSpec((B, 1), lambda e, j: (0, 0)),
        ],
        out_specs=pl.BlockSpec((B, 1), lambda e, j: (0, 0)),
        out_shape=jax.ShapeDtypeStruct((B, 1), jnp.float32),
        compiler_params=pltpu.CompilerParams(
            dimension_semantics=("arbitrary", "arbitrary")),
    )(xb, W1, b1r, W2r, gates, y0)
    return out
